# baseline (speedup 1.0000x reference)
import jax
import jax.numpy as jnp
from jax import lax
from jax.experimental import pallas as pl
from jax.experimental.pallas import tpu as pltpu
from jax.experimental.pallas import tpu_sc as plsc

_INFO = plsc.get_sparse_core_info()
_NC = _INFO.num_cores
_NS = _INFO.num_subcores
_NW = _NC * _NS

_B = 1024
_S = 200
_D = 128
_EPS = 1e-5

_TOK = _B * _S
_PER_W = _TOK // _NW
_CHUNK = 128
_NCHUNK = _PER_W // _CHUNK
_V8 = _D // 16


def _rsqrt_vec(x):
    i = lax.bitcast_convert_type(x, jnp.int32)
    i = jnp.int32(0x5F3759DF) - lax.shift_right_logical(i, 1)
    y = lax.bitcast_convert_type(i, jnp.float32)
    for _ in range(3):
        y = y * (1.5 - 0.5 * x * y * y)
    return y


def _body(ids_hbm, word_hbm, pos_hbm, gamma_hbm, beta_hbm, out_hbm,
          idx_v, pos_v, g_v, b_v, r0, r1, o0, o1, sg0, sg1, ss0, ss1):
    cid = lax.axis_index("c")
    sid = lax.axis_index("s")
    wid = sid * _NC + cid

    pltpu.sync_copy(ids_hbm.at[wid], idx_v)
    pltpu.sync_copy(pos_hbm.at[pl.ds(0, _S)], pos_v)
    pltpu.sync_copy(gamma_hbm, g_v)
    pltpu.sync_copy(beta_hbm, b_v)

    pltpu.async_copy(word_hbm.at[idx_v.at[0]], r0, sg0)
    pltpu.async_copy(word_hbm.at[idx_v.at[1]], r1, sg1)

    lanes = lax.iota(jnp.int32, 16)
    inv_d = jnp.float32(1.0 / _D)
    g_r = [g_v[pl.ds(16 * v, 16)] for v in range(_V8)]
    b_r = [b_v[pl.ds(16 * v, 16)] for v in range(_V8)]

    def compute(c, r, o):
        @plsc.parallel_loop(0, _CHUNK // 16, unroll=1)
        def grp(g):
            j0 = g * 16
            ids16 = idx_v[c, pl.ds(j0, 16)]
            m = jnp.where(ids16 == jnp.int32(0),
                          jnp.float32(0.0), jnp.float32(1.0))
            tok = lanes + j0
            pbase = lax.rem(c * _CHUNK + j0, jnp.int32(_S))
            p = pbase + lanes
            p = jnp.where(p >= _S, p - _S, p)

            zf = jnp.zeros((16,), jnp.float32)

            @plsc.parallel_loop(0, _D, carry=(zf, zf), unroll=8)
            def p1(f, acc):
                s, q = acc
                fv = f + lanes
                fv = jnp.where(fv >= _D, fv - _D, fv)
                w = plsc.load_gather(r, [tok, fv])
                pe = plsc.load_gather(pos_v, [p, fv])
                x = w * m + pe
                plsc.store_scatter(r, [tok, fv], x)
                return (s + x, q + x * x)

            s, q = p1
            mean = s * inv_d
            var = q * inv_d - mean * mean
            rs = _rsqrt_vec(var + _EPS)

            for j in range(16):
                ms = mean[j]
                rss = rs[j]
                row = j0 + j
                for v in range(_V8):
                    a = g_r[v] * rss
                    bb = b_r[v] - a * ms
                    xv = r[row, pl.ds(16 * v, 16)]
                    o[row, pl.ds(16 * v, 16)] = xv * a + bb

    def outer(i, carry):
        c0 = 2 * i
        for b, (r, o, sg, ss) in enumerate(
                ((r0, o0, sg0, ss0), (r1, o1, sg1, ss1))):
            c = c0 + b
            pltpu.make_async_copy(word_hbm.at[pl.ds(0, _CHUNK)], r, sg).wait()

            @pl.when(c >= 2)
            def _():
                pltpu.make_async_copy(o, out_hbm.at[0], ss).wait()

            compute(c, r, o)
            pltpu.async_copy(o, out_hbm.at[wid * _NCHUNK + c], ss)

            @pl.when(c + 2 < _NCHUNK)
            def _():
                pltpu.async_copy(word_hbm.at[idx_v.at[c + 2]], r, sg)
        return carry

    lax.fori_loop(0, _NCHUNK // 2, outer, None)

    pltpu.make_async_copy(o0, out_hbm.at[0], ss0).wait()
    pltpu.make_async_copy(o1, out_hbm.at[0], ss1).wait()


_emb_ln = pl.kernel(
    _body,
    out_type=jax.ShapeDtypeStruct((_NW * _NCHUNK, _CHUNK, _D), jnp.float32),
    mesh=plsc.VectorSubcoreMesh(core_axis_name="c", subcore_axis_name="s"),
    compiler_params=pltpu.CompilerParams(needs_layout_passes=False),
    scratch_types=[
        pltpu.VMEM((_NCHUNK, _CHUNK), jnp.int32),
        pltpu.VMEM((_S, _D), jnp.float32),
        pltpu.VMEM((_D,), jnp.float32),
        pltpu.VMEM((_D,), jnp.float32),
        pltpu.VMEM((_CHUNK, _D), jnp.float32),
        pltpu.VMEM((_CHUNK, _D), jnp.float32),
        pltpu.VMEM((_CHUNK, _D), jnp.float32),
        pltpu.VMEM((_CHUNK, _D), jnp.float32),
        pltpu.SemaphoreType.DMA,
        pltpu.SemaphoreType.DMA,
        pltpu.SemaphoreType.DMA,
        pltpu.SemaphoreType.DMA,
    ],
)


def kernel(input_ids, word_table, pos_table, gamma, beta):
    B, S = input_ids.shape
    ids3 = input_ids.astype(jnp.int32).reshape(_NW, _NCHUNK, _CHUNK)
    out = _emb_ln(ids3, word_table, pos_table, gamma, beta)
    return out.reshape(B, S, _D)


# speedup vs baseline: 1.0163x; 1.0163x over previous
import jax
import jax.numpy as jnp
from jax import lax
from jax.experimental import pallas as pl
from jax.experimental.pallas import tpu as pltpu
from jax.experimental.pallas import tpu_sc as plsc

_INFO = plsc.get_sparse_core_info()
_NC = _INFO.num_cores
_NS = _INFO.num_subcores
_NW = _NC * _NS

_B = 1024
_S = 200
_D = 128
_EPS = 1e-5

_TOK = _B * _S
_PER_W = _TOK // _NW
_CHUNK = 128
_NCHUNK = _PER_W // _CHUNK
_V8 = _D // 16


def _rsqrt_vec(x):
    i = lax.bitcast_convert_type(x, jnp.int32)
    i = jnp.int32(0x5F3759DF) - lax.shift_right_logical(i, 1)
    y = lax.bitcast_convert_type(i, jnp.float32)
    for _ in range(3):
        y = y * (1.5 - 0.5 * x * y * y)
    return y


def _body(ids_hbm, word_hbm, pos_hbm, gamma_hbm, beta_hbm, out_hbm,
          idx_v, pos_v, g_v, b_v, r0, r1, o0, o1, sg0, sg1, ss0, ss1):
    cid = lax.axis_index("c")
    sid = lax.axis_index("s")
    wid = sid * _NC + cid

    pltpu.sync_copy(ids_hbm.at[wid], idx_v)
    pltpu.sync_copy(pos_hbm.at[pl.ds(0, _S)], pos_v)
    pltpu.sync_copy(gamma_hbm, g_v)
    pltpu.sync_copy(beta_hbm, b_v)

    pltpu.async_copy(word_hbm.at[idx_v.at[0]], r0, sg0)
    pltpu.async_copy(word_hbm.at[idx_v.at[1]], r1, sg1)

    lanes = lax.iota(jnp.int32, 16)
    inv_d = jnp.float32(1.0 / _D)
    g_r = [g_v[pl.ds(16 * v, 16)] for v in range(_V8)]
    b_r = [b_v[pl.ds(16 * v, 16)] for v in range(_V8)]

    def compute(c, r, o):
        @plsc.parallel_loop(0, _CHUNK // 16, unroll=1)
        def grp(g):
            j0 = g * 16
            ids16 = idx_v[c, pl.ds(j0, 16)]
            m = jnp.where(ids16 == jnp.int32(0),
                          jnp.float32(0.0), jnp.float32(1.0))
            tok = lanes + j0
            pbase = lax.rem(c * _CHUNK + j0, jnp.int32(_S))
            p = pbase + lanes
            p = jnp.where(p >= _S, p - _S, p)

            zf = jnp.zeros((16,), jnp.float32)

            @plsc.parallel_loop(0, _D, carry=(zf, zf), unroll=16)
            def p1(f, acc):
                s, q = acc
                fv = f + lanes
                fv = jnp.where(fv >= _D, fv - _D, fv)
                w = plsc.load_gather(r, [tok, fv])
                pe = plsc.load_gather(pos_v, [p, fv])
                x = w * m + pe
                plsc.store_scatter(r, [tok, fv], x)
                return (s + x, q + x * x)

            s, q = p1
            mean = s * inv_d
            var = q * inv_d - mean * mean
            rs = _rsqrt_vec(var + _EPS)

            for j in range(16):
                ms = mean[j]
                rss = rs[j]
                row = j0 + j
                for v in range(_V8):
                    a = g_r[v] * rss
                    bb = b_r[v] - a * ms
                    xv = r[row, pl.ds(16 * v, 16)]
                    o[row, pl.ds(16 * v, 16)] = xv * a + bb

    def outer(i, carry):
        c0 = 2 * i
        for b, (r, o, sg, ss) in enumerate(
                ((r0, o0, sg0, ss0), (r1, o1, sg1, ss1))):
            c = c0 + b
            pltpu.make_async_copy(word_hbm.at[pl.ds(0, _CHUNK)], r, sg).wait()

            @pl.when(c >= 2)
            def _():
                pltpu.make_async_copy(o, out_hbm.at[0], ss).wait()

            compute(c, r, o)
            pltpu.async_copy(o, out_hbm.at[wid * _NCHUNK + c], ss)

            @pl.when(c + 2 < _NCHUNK)
            def _():
                pltpu.async_copy(word_hbm.at[idx_v.at[c + 2]], r, sg)
        return carry

    lax.fori_loop(0, _NCHUNK // 2, outer, None)

    pltpu.make_async_copy(o0, out_hbm.at[0], ss0).wait()
    pltpu.make_async_copy(o1, out_hbm.at[0], ss1).wait()


_emb_ln = pl.kernel(
    _body,
    out_type=jax.ShapeDtypeStruct((_NW * _NCHUNK, _CHUNK, _D), jnp.float32),
    mesh=plsc.VectorSubcoreMesh(core_axis_name="c", subcore_axis_name="s"),
    compiler_params=pltpu.CompilerParams(needs_layout_passes=False),
    scratch_types=[
        pltpu.VMEM((_NCHUNK, _CHUNK), jnp.int32),
        pltpu.VMEM((_S, _D), jnp.float32),
        pltpu.VMEM((_D,), jnp.float32),
        pltpu.VMEM((_D,), jnp.float32),
        pltpu.VMEM((_CHUNK, _D), jnp.float32),
        pltpu.VMEM((_CHUNK, _D), jnp.float32),
        pltpu.VMEM((_CHUNK, _D), jnp.float32),
        pltpu.VMEM((_CHUNK, _D), jnp.float32),
        pltpu.SemaphoreType.DMA,
        pltpu.SemaphoreType.DMA,
        pltpu.SemaphoreType.DMA,
        pltpu.SemaphoreType.DMA,
    ],
)


def kernel(input_ids, word_table, pos_table, gamma, beta):
    B, S = input_ids.shape
    ids3 = input_ids.astype(jnp.int32).reshape(_NW, _NCHUNK, _CHUNK)
    out = _emb_ln(ids3, word_table, pos_table, gamma, beta)
    return out.reshape(B, S, _D)
